# baseline (speedup 1.0000x reference)
import functools

import jax
import jax.numpy as jnp
from jax import lax
from jax.experimental import pallas as pl
from jax.experimental.pallas import tpu as pltpu
from jax.experimental.pallas import tpu_sc as plsc

V = 1000
VP = 1024
B = 1024
T = 50
N = B * T

NC = 2
NS = 16
NW = NC * NS
ROWS_PER_W = N // NW
CHUNK = T
TPAD = 56
BATCHES_PER_W = B // NW
L = 16
PG = 80
NPG = ROWS_PER_W // PG


def _lse_body(table_ref, lse_ref):
    t = table_ref[...]
    m = jnp.max(t, axis=1)
    s = jnp.sum(jnp.exp(t - m[:, None]), axis=1)
    lse_ref[...] = m + jnp.log(s)


def _row_lse(table):
    return pl.pallas_call(
        _lse_body,
        out_shape=jax.ShapeDtypeStruct((V,), jnp.float32),
    )(table)


_sc_mesh = plsc.VectorSubcoreMesh(core_axis_name="c", subcore_axis_name="s")


@functools.partial(
    pl.kernel,
    mesh=_sc_mesh,
    out_type=jax.ShapeDtypeStruct((B, TPAD, VP), jnp.float32),
    scratch_types=[
        pltpu.VMEM((BATCHES_PER_W * TPAD,), jnp.int32),
        pltpu.VMEM((TPAD, VP), jnp.float32),
        pltpu.VMEM((TPAD, VP), jnp.float32),
        pltpu.SemaphoreType.DMA,
        pltpu.SemaphoreType.DMA,
        pltpu.SemaphoreType.DMA,
    ],
)
def _sc_rows(x_hbm, tablep_hbm, out_hbm,
             xv, rows0, rows1, gsem, ssem0, ssem1):
    bufs = (rows0, rows1)
    ssems = (ssem0, ssem1)
    wid = lax.axis_index("s") * NC + lax.axis_index("c")
    base = wid * BATCHES_PER_W
    pltpu.sync_copy(
        x_hbm.at[pl.ds(base * TPAD, BATCHES_PER_W * TPAD)], xv)

    def scatter_issue(k, b):
        pltpu.async_copy(bufs[b], out_hbm.at[base + k], ssems[b])

    def scatter_wait(b):
        pltpu.make_async_copy(bufs[b], out_hbm.at[0], ssems[b]).wait()

    def outer_body(o, carry):
        for b in range(2):
            k = 2 * o + b

            @pl.when(o >= 1)
            def _():
                scatter_wait(b)
            pltpu.async_copy(
                tablep_hbm.at[xv.at[pl.ds(k * TPAD, TPAD)]],
                bufs[b], gsem).wait()
            scatter_issue(k, b)
        return carry

    lax.fori_loop(0, BATCHES_PER_W // 2, outer_body, 0)
    scatter_wait(0)
    scatter_wait(1)


@functools.partial(
    pl.kernel,
    mesh=_sc_mesh,
    out_type=jax.ShapeDtypeStruct((NW * L,), jnp.float32),
    scratch_types=[
        pltpu.VMEM((ROWS_PER_W,), jnp.int32),
        pltpu.VMEM((ROWS_PER_W,), jnp.int32),
        pltpu.VMEM((ROWS_PER_W,), jnp.int32),
        pltpu.VMEM((ROWS_PER_W,), jnp.float32),
        pltpu.VMEM((ROWS_PER_W,), jnp.float32),
        pltpu.VMEM((L,), jnp.float32),
        pltpu.SemaphoreType.DMA,
    ],
)
def _sc_loss(x_hbm, tgt_hbm, lse_hbm, tflat_hbm, loss_hbm,
             xv, tv, pidxv, pickedv, lsexv, accv, psem):
    wid = lax.axis_index("s") * NC + lax.axis_index("c")
    base = wid * ROWS_PER_W
    pltpu.sync_copy(x_hbm.at[pl.ds(base, ROWS_PER_W)], xv)
    pltpu.sync_copy(tgt_hbm.at[pl.ds(base, ROWS_PER_W)], tv)

    def pidx_body(g, carry):
        xg = xv[pl.ds(g * L, L)]
        tg = tv[pl.ds(g * L, L)]
        pidxv[pl.ds(g * L, L)] = xg * V + tg
        return carry

    lax.fori_loop(0, ROWS_PER_W // L, pidx_body, 0)

    for k in range(NPG):
        pltpu.async_copy(
            tflat_hbm.at[pidxv.at[pl.ds(k * PG, PG)]],
            pickedv.at[pl.ds(k * PG, PG)], psem)
        pltpu.async_copy(
            lse_hbm.at[xv.at[pl.ds(k * PG, PG)]],
            lsexv.at[pl.ds(k * PG, PG)], psem)
    for k in range(NPG):
        pltpu.make_async_copy(
            tflat_hbm.at[pl.ds(0, PG)], pickedv.at[pl.ds(0, PG)], psem).wait()
        pltpu.make_async_copy(
            lse_hbm.at[pl.ds(0, PG)], lsexv.at[pl.ds(0, PG)], psem).wait()

    def acc_body(g, acc):
        return acc + (lsexv[pl.ds(g * L, L)] - pickedv[pl.ds(g * L, L)])

    acc = lax.fori_loop(0, ROWS_PER_W // L, acc_body,
                        jnp.zeros((L,), jnp.float32))
    accv[...] = acc
    pltpu.sync_copy(accv, loss_hbm.at[pl.ds(wid * L, L)])


def kernel(x, target, table):
    lse = _row_lse(table)
    tablep = jnp.pad(table, ((0, 0), (0, VP - V)))
    xp = jnp.pad(x, ((0, 0), (0, TPAD - T))).reshape(-1)
    logits_pad = _sc_rows(xp, tablep)
    logits = logits_pad[:, :T, :V]
    loss_parts = _sc_loss(
        x.reshape(-1), target.reshape(-1), lse, table.reshape(-1))
    loss = jnp.sum(loss_parts) / jnp.float32(N)
    return (logits, loss)


# speedup vs baseline: 1.2008x; 1.2008x over previous
import functools

import jax
import jax.numpy as jnp
from jax import lax
from jax.experimental import pallas as pl
from jax.experimental.pallas import tpu as pltpu
from jax.experimental.pallas import tpu_sc as plsc

V = 1000
B = 1024
T = 50
N = B * T
TPAD = 64

NC = 2
NS = 16
NW = NC * NS
BATCHES_PER_W = B // NW
L = 16
NG = (T + L - 1) // L


def _lse_body(table_ref, lse_ref):
    t = table_ref[...]
    m = jnp.max(t, axis=1)
    s = jnp.sum(jnp.exp(t - m[:, None]), axis=1)
    lse_ref[...] = m + jnp.log(s)


def _row_lse(table):
    return pl.pallas_call(
        _lse_body,
        out_shape=jax.ShapeDtypeStruct((V,), jnp.float32),
    )(table)


_sc_mesh = plsc.VectorSubcoreMesh(core_axis_name="c", subcore_axis_name="s")


@functools.partial(
    pl.kernel,
    mesh=_sc_mesh,
    compiler_params=pltpu.CompilerParams(
        use_tc_tiling_on_sc=False, needs_layout_passes=False),
    out_type=(
        jax.ShapeDtypeStruct((B, T, V), jnp.float32),
        jax.ShapeDtypeStruct((NW * L,), jnp.float32),
    ),
    scratch_types=[
        pltpu.VMEM((BATCHES_PER_W * TPAD,), jnp.int32),
        pltpu.VMEM((BATCHES_PER_W * TPAD,), jnp.int32),
        pltpu.VMEM((V,), jnp.float32),
        pltpu.VMEM((T, V), jnp.float32),
        pltpu.VMEM((T, V), jnp.float32),
        pltpu.VMEM((L,), jnp.float32),
        pltpu.SemaphoreType.DMA,
        pltpu.SemaphoreType.DMA,
        pltpu.SemaphoreType.DMA,
        pltpu.SemaphoreType.DMA,
    ],
)
def _sc_main(xp_hbm, tp_hbm, lse_hbm, table_hbm, out_hbm, loss_hbm,
             xv, tv, lsev, rows0, rows1, accv,
             gsem0, gsem1, ssem0, ssem1):
    bufs = (rows0, rows1)
    gsems = (gsem0, gsem1)
    ssems = (ssem0, ssem1)
    wid = lax.axis_index("s") * NC + lax.axis_index("c")
    base = wid * BATCHES_PER_W
    pltpu.sync_copy(
        xp_hbm.at[pl.ds(base * TPAD, BATCHES_PER_W * TPAD)], xv)
    pltpu.sync_copy(
        tp_hbm.at[pl.ds(base * TPAD, BATCHES_PER_W * TPAD)], tv)
    pltpu.sync_copy(lse_hbm, lsev)

    def gather_issue(k, b):
        pltpu.async_copy(
            table_hbm.at[xv.at[pl.ds(k * TPAD, T)]], bufs[b], gsems[b])

    def gather_wait(b):
        pltpu.make_async_copy(
            table_hbm.at[pl.ds(0, T)], bufs[b], gsems[b]).wait()

    def scatter_issue(k, b):
        pltpu.async_copy(bufs[b], out_hbm.at[base + k], ssems[b])

    def scatter_wait(b):
        pltpu.make_async_copy(bufs[b], out_hbm.at[0], ssems[b]).wait()

    def chunk_loss(k, b, acc):
        for g in range(NG):
            sl = pl.ds(k * TPAD + g * L, L)
            xg = xv[sl]
            tg = tv[sl]
            row0 = g * L
            if row0 + L <= T:
                row_ids = lax.iota(jnp.int32, L) + row0
                picked = plsc.load_gather(bufs[b], [row_ids, tg])
                lx = plsc.load_gather(lsev, [xg])
                acc = acc + (lx - picked)
            else:
                ri = lax.iota(jnp.int32, L)
                mask = ri < (T - row0)
                row_ids = jnp.minimum(ri + row0, T - 1)
                picked = plsc.load_gather(bufs[b], [row_ids, tg])
                lx = plsc.load_gather(lsev, [xg])
                acc = acc + jnp.where(mask, lx - picked,
                                      jnp.zeros((L,), jnp.float32))
        return acc

    gather_issue(0, 0)

    def outer_body(o, acc):
        for b in range(2):
            k = 2 * o + b
            nb = 1 - b
            if b == 0:
                @pl.when(o >= 1)
                def _():
                    scatter_wait(nb)
                gather_issue(k + 1, nb)
            else:
                scatter_wait(nb)

                @pl.when(o <= BATCHES_PER_W // 2 - 2)
                def _():
                    gather_issue(k + 1, nb)
            gather_wait(b)
            acc = chunk_loss(k, b, acc)
            scatter_issue(k, b)
        return acc

    acc = lax.fori_loop(0, BATCHES_PER_W // 2, outer_body,
                        jnp.zeros((L,), jnp.float32))
    scatter_wait(1)
    accv[...] = acc
    pltpu.sync_copy(accv, loss_hbm.at[pl.ds(wid * L, L)])


def kernel(x, target, table):
    lse = _row_lse(table)
    xp = jnp.pad(x, ((0, 0), (0, TPAD - T))).reshape(-1)
    tp = jnp.pad(target, ((0, 0), (0, TPAD - T))).reshape(-1)
    logits, loss_parts = _sc_main(xp, tp, lse, table)
    loss = jnp.sum(loss_parts) / jnp.float32(N)
    return (logits, loss)
